# Initial kernel scaffold; baseline (speedup 1.0000x reference)
import jax
import jax.numpy as jnp
from jax.experimental import pallas as pl


def kernel(fc_adj, sc_adj, fc_W1, fc_b1, fc_W2, fc_b2, sc_W1, sc_b1, sc_W2, sc_b2, head_W1, head_b1, head_W2, head_b2):
    raise NotImplementedError("write your pallas kernel here")


# speedup vs baseline: 3368.0612x; 3368.0612x over previous
import jax
import jax.numpy as jnp
from jax.experimental import pallas as pl

N = 512
B = 4
HID = 64
EMB = 128
NC = 2


def _encode_body(adj_ref, w1_ref, b1_ref, w2_ref, b2_ref, out_ref):
    a = adj_ref[0]
    row = jax.lax.broadcasted_iota(jnp.int32, (N, N), 0)
    col = jax.lax.broadcasted_iota(jnp.int32, (N, N), 1)
    up = jnp.where(col > row, a, 0.0)
    abar = up + up.T
    deg = jnp.sum(abar, axis=1) + 1.0
    dinv = jnp.where(deg > 0, jax.lax.rsqrt(deg), 0.0)
    eye = jnp.where(row == col, 1.0, 0.0)
    ahat = (abar + eye) * (dinv[:, None] * dinv[None, :])

    h0 = jax.lax.dot_general(a, w1_ref[...], (((1,), (1,)), ((), ())),
                             preferred_element_type=jnp.float32)
    h1 = jnp.maximum(
        jnp.dot(ahat, h0, preferred_element_type=jnp.float32) + b1_ref[0], 0.0)
    g0 = jax.lax.dot_general(h1, w2_ref[...], (((1,), (1,)), ((), ())),
                             preferred_element_type=jnp.float32)
    h2 = jnp.maximum(
        jnp.dot(ahat, g0, preferred_element_type=jnp.float32) + b2_ref[0], 0.0)
    out_ref[0, 0, :] = jnp.mean(h2, axis=0)


def _encode(adj, W1, b1, W2, b2):
    return pl.pallas_call(
        _encode_body,
        grid=(B,),
        in_specs=[
            pl.BlockSpec((1, N, N), lambda b: (b, 0, 0)),
            pl.BlockSpec((HID, N), lambda b: (0, 0)),
            pl.BlockSpec((1, HID), lambda b: (0, 0)),
            pl.BlockSpec((EMB, HID), lambda b: (0, 0)),
            pl.BlockSpec((1, EMB), lambda b: (0, 0)),
        ],
        out_specs=pl.BlockSpec((1, 1, EMB), lambda b: (b, 0, 0)),
        out_shape=jax.ShapeDtypeStruct((B, 1, EMB), jnp.float32),
    )(adj, W1, b1.reshape(1, HID), W2, b2.reshape(1, EMB)).reshape(B, EMB)


def _head_body(feat_ref, w1_ref, b1_ref, w2_ref, b2_ref, out_ref):
    h = jnp.maximum(
        jax.lax.dot_general(feat_ref[...], w1_ref[...], (((1,), (1,)), ((), ())),
                            preferred_element_type=jnp.float32) + b1_ref[0], 0.0)
    out_ref[...] = jax.lax.dot_general(
        h, w2_ref[...], (((1,), (1,)), ((), ())),
        preferred_element_type=jnp.float32) + b2_ref[0]


def _head(feat, W1, b1, W2, b2):
    return pl.pallas_call(
        _head_body,
        out_shape=jax.ShapeDtypeStruct((B, NC), jnp.float32),
    )(feat, W1, b1.reshape(1, 2 * HID), W2, b2.reshape(1, NC))


def kernel(fc_adj, sc_adj, fc_W1, fc_b1, fc_W2, fc_b2,
           sc_W1, sc_b1, sc_W2, sc_b2, head_W1, head_b1, head_W2, head_b2):
    fc_emb = _encode(fc_adj, fc_W1, fc_b1, fc_W2, fc_b2)
    sc_emb = _encode(sc_adj, sc_W1, sc_b1, sc_W2, sc_b2)
    feat = jnp.concatenate([fc_emb, sc_emb], axis=1)
    return _head(feat, head_W1, head_b1, head_W2, head_b2)


# speedup vs baseline: 4426.1849x; 1.3142x over previous
import jax
import jax.numpy as jnp
from jax.experimental import pallas as pl
from jax.experimental.pallas import tpu as pltpu

N = 512
B = 4
HID = 64
EMB = 128
NC = 2


def _bf16_dot(a, b):
    return jnp.dot(a.astype(jnp.bfloat16), b.astype(jnp.bfloat16),
                   preferred_element_type=jnp.float32)


def _bf16_dot_t(a, b):
    return jax.lax.dot_general(
        a.astype(jnp.bfloat16), b.astype(jnp.bfloat16),
        (((1,), (1,)), ((), ())), preferred_element_type=jnp.float32)


def _encode_one(a, w1, b1, w2, b2, row, col):
    up = jnp.where(col > row, a, 0.0)
    abar = up + up.T
    deg = jnp.sum(abar, axis=1) + 1.0
    dinv = jnp.where(deg > 0, jax.lax.rsqrt(deg), 0.0)[:, None]

    abar16 = abar.astype(jnp.bfloat16)
    h0 = _bf16_dot_t(a, w1)
    t1 = h0 * dinv
    s1 = _bf16_dot(abar16, t1) + t1
    h1 = jnp.maximum(s1 * dinv + b1, 0.0)

    g0 = _bf16_dot_t(h1, w2)
    t2 = g0 * dinv
    s2 = _bf16_dot(abar16, t2) + t2
    h2 = jnp.maximum(s2 * dinv + b2, 0.0)
    return jnp.mean(h2, axis=0)


def _fused_body(fc_ref, sc_ref, fw1_ref, fb1_ref, fw2_ref, fb2_ref,
                sw1_ref, sb1_ref, sw2_ref, sb2_ref,
                hw1_ref, hb1_ref, hw2_ref, hb2_ref, out_ref, feat_ref):
    b = pl.program_id(0)
    row = jax.lax.broadcasted_iota(jnp.int32, (N, N), 0)
    col = jax.lax.broadcasted_iota(jnp.int32, (N, N), 1)

    fc_emb = _encode_one(fc_ref[0], fw1_ref[...], fb1_ref[0], fw2_ref[...],
                         fb2_ref[0], row, col)
    sc_emb = _encode_one(sc_ref[0], sw1_ref[...], sb1_ref[0], sw2_ref[...],
                         sb2_ref[0], row, col)
    feat_ref[pl.ds(b, 1), :] = jnp.concatenate([fc_emb, sc_emb])[None, :]

    @pl.when(b == B - 1)
    def _():
        feat = feat_ref[...]
        h = jnp.maximum(
            jax.lax.dot_general(feat, hw1_ref[...], (((1,), (1,)), ((), ())),
                                preferred_element_type=jnp.float32)
            + hb1_ref[0], 0.0)
        out_ref[...] = jax.lax.dot_general(
            h, hw2_ref[...], (((1,), (1,)), ((), ())),
            preferred_element_type=jnp.float32) + hb2_ref[0]


def kernel(fc_adj, sc_adj, fc_W1, fc_b1, fc_W2, fc_b2,
           sc_W1, sc_b1, sc_W2, sc_b2, head_W1, head_b1, head_W2, head_b2):
    full = lambda shape: pl.BlockSpec(shape, lambda b: tuple(0 for _ in shape))
    return pl.pallas_call(
        _fused_body,
        grid=(B,),
        in_specs=[
            pl.BlockSpec((1, N, N), lambda b: (b, 0, 0)),
            pl.BlockSpec((1, N, N), lambda b: (b, 0, 0)),
            full((HID, N)), full((1, HID)), full((EMB, HID)), full((1, EMB)),
            full((HID, N)), full((1, HID)), full((EMB, HID)), full((1, EMB)),
            full((2 * HID, 2 * EMB)), full((1, 2 * HID)),
            full((NC, 2 * HID)), full((1, NC)),
        ],
        out_specs=pl.BlockSpec((B, NC), lambda b: (0, 0)),
        out_shape=jax.ShapeDtypeStruct((B, NC), jnp.float32),
        scratch_shapes=[pltpu.VMEM((B, 2 * EMB), jnp.float32)],
    )(fc_adj, sc_adj,
      fc_W1, fc_b1.reshape(1, HID), fc_W2, fc_b2.reshape(1, EMB),
      sc_W1, sc_b1.reshape(1, HID), sc_W2, sc_b2.reshape(1, EMB),
      head_W1, head_b1.reshape(1, 2 * HID), head_W2, head_b2.reshape(1, NC))


# speedup vs baseline: 4455.4205x; 1.0066x over previous
import jax
import jax.numpy as jnp
from jax.experimental import pallas as pl
from jax.experimental.pallas import tpu as pltpu

N = 512
B = 4
HID = 64
EMB = 128
NC = 2


def _bf16_dot(a, b):
    return jnp.dot(a.astype(jnp.bfloat16), b.astype(jnp.bfloat16),
                   preferred_element_type=jnp.float32)


def _bf16_dot_t(a, b):
    return jax.lax.dot_general(
        a.astype(jnp.bfloat16), b.astype(jnp.bfloat16),
        (((1,), (1,)), ((), ())), preferred_element_type=jnp.float32)


def _encode_one(a, w1, b1, w2, b2, row, col):
    up = jnp.where(col > row, a, 0.0)
    abar = up + up.T
    deg = jnp.sum(abar, axis=1) + 1.0
    dinv = jax.lax.rsqrt(deg)[:, None]

    abar16 = abar.astype(jnp.bfloat16)
    h0 = _bf16_dot_t(a, w1)
    t1 = h0 * dinv
    s1 = _bf16_dot(abar16, t1) + t1
    h1 = jnp.maximum(s1 * dinv + b1, 0.0)

    g0 = _bf16_dot_t(h1, w2)
    t2 = g0 * dinv
    s2 = _bf16_dot(abar16, t2) + t2
    h2 = jnp.maximum(s2 * dinv + b2, 0.0)
    return jnp.mean(h2, axis=0)


def _fused_body(fc_ref, sc_ref, fw1_ref, fb1_ref, fw2_ref, fb2_ref,
                sw1_ref, sb1_ref, sw2_ref, sb2_ref,
                hw1_ref, hb1_ref, hw2_ref, hb2_ref, out_ref, feat_ref):
    b = pl.program_id(0)
    row = jax.lax.broadcasted_iota(jnp.int32, (N, N), 0)
    col = jax.lax.broadcasted_iota(jnp.int32, (N, N), 1)

    fc_emb = _encode_one(fc_ref[0], fw1_ref[...], fb1_ref[0], fw2_ref[...],
                         fb2_ref[0], row, col)
    sc_emb = _encode_one(sc_ref[0], sw1_ref[...], sb1_ref[0], sw2_ref[...],
                         sb2_ref[0], row, col)
    feat_ref[pl.ds(b, 1), :] = jnp.concatenate([fc_emb, sc_emb])[None, :]

    @pl.when(b == B - 1)
    def _():
        feat = feat_ref[...]
        h = jnp.maximum(_bf16_dot_t(feat, hw1_ref[...]) + hb1_ref[0], 0.0)
        out_ref[...] = _bf16_dot_t(h, hw2_ref[...]) + hb2_ref[0]


def kernel(fc_adj, sc_adj, fc_W1, fc_b1, fc_W2, fc_b2,
           sc_W1, sc_b1, sc_W2, sc_b2, head_W1, head_b1, head_W2, head_b2):
    full = lambda shape: pl.BlockSpec(shape, lambda b: tuple(0 for _ in shape))
    return pl.pallas_call(
        _fused_body,
        grid=(B,),
        in_specs=[
            pl.BlockSpec((1, N, N), lambda b: (b, 0, 0)),
            pl.BlockSpec((1, N, N), lambda b: (b, 0, 0)),
            full((HID, N)), full((1, HID)), full((EMB, HID)), full((1, EMB)),
            full((HID, N)), full((1, HID)), full((EMB, HID)), full((1, EMB)),
            full((2 * HID, 2 * EMB)), full((1, 2 * HID)),
            full((NC, 2 * HID)), full((1, NC)),
        ],
        out_specs=pl.BlockSpec((B, NC), lambda b: (0, 0)),
        out_shape=jax.ShapeDtypeStruct((B, NC), jnp.float32),
        scratch_shapes=[pltpu.VMEM((B, 2 * EMB), jnp.float32)],
    )(fc_adj, sc_adj,
      fc_W1, fc_b1.reshape(1, HID), fc_W2, fc_b2.reshape(1, EMB),
      sc_W1, sc_b1.reshape(1, HID), sc_W2, sc_b2.reshape(1, EMB),
      head_W1, head_b1.reshape(1, 2 * HID), head_W2, head_b2.reshape(1, NC))
